# baseline (speedup 1.0000x reference)
import functools

import jax
import jax.numpy as jnp
from jax import lax
from jax.experimental import pallas as pl
from jax.experimental.pallas import tpu as pltpu
from jax.experimental.pallas import tpu_sc as plsc

L = 16
NUM_CORES = 2
NUM_SUBCORES = 16
NW = NUM_CORES * NUM_SUBCORES
B = 2
S = 8192
HD2 = 64
D_PER_W = HD2 * B // NW

_mesh = plsc.VectorSubcoreMesh(core_axis_name="c", subcore_axis_name="s")


UNROLL = 16
QUARTERS = 2
S_Q = S // QUARTERS


@functools.partial(
    pl.kernel,
    mesh=_mesh,
    out_type=jax.ShapeDtypeStruct((B, HD2, S), jnp.float32),
    scratch_types=[
        pltpu.VMEM((S,), jnp.int32),
        pltpu.VMEM((2 * HD2,), jnp.float32),
        pltpu.VMEM((D_PER_W, S), jnp.float32),
        pltpu.SemaphoreType.DMA,
        pltpu.SemaphoreType.DMA,
        pltpu.SemaphoreType.DMA,
    ],
)
def _rope_theta_sc(pos_hbm, invf_hbm, out_hbm, pos_v, invf_v, out_v,
                   in_sem, invf_sem, out_sem):
    wid = lax.axis_index("s") * NUM_CORES + lax.axis_index("c")
    b = wid // NUM_SUBCORES
    wd = wid % NUM_SUBCORES
    d0 = wd * D_PER_W
    h_pos = pltpu.async_copy(pos_hbm.at[b], pos_v, in_sem)
    h_invf = pltpu.async_copy(invf_hbm, invf_v.at[pl.ds(0, HD2)], invf_sem)
    h_invf.wait()
    h_pos.wait()
    fvr = invf_v[pl.ds(d0, L)]
    fs = [fvr[dd] for dd in range(D_PER_W)]

    out_handles = []
    for q in range(QUARTERS):
        @plsc.parallel_loop(q * S_Q, (q + 1) * S_Q, step=L, unroll=UNROLL)
        def body(row):
            pf = pos_v[pl.ds(row, L)].astype(jnp.float32)
            for dd in range(D_PER_W):
                out_v[dd, pl.ds(row, L)] = pf * fs[dd]

        out_handles.append(pltpu.async_copy(
            out_v.at[:, pl.ds(q * S_Q, S_Q)],
            out_hbm.at[b, pl.ds(d0, D_PER_W), pl.ds(q * S_Q, S_Q)],
            out_sem))
    for h in out_handles:
        h.wait()


def kernel(position_ids, inv_freq):
    out_t = _rope_theta_sc(position_ids.astype(jnp.int32), inv_freq)
    return jnp.swapaxes(out_t, 1, 2)


# speedup vs baseline: 1.0090x; 1.0090x over previous
import functools

import jax
import jax.numpy as jnp
from jax import lax
from jax.experimental import pallas as pl
from jax.experimental.pallas import tpu as pltpu
from jax.experimental.pallas import tpu_sc as plsc

L = 16
NUM_CORES = 2
NUM_SUBCORES = 16
NW = NUM_CORES * NUM_SUBCORES
B = 2
S = 8192
HD2 = 64
D_PER_W = HD2 * B // NW

_mesh = plsc.VectorSubcoreMesh(core_axis_name="c", subcore_axis_name="s")


UNROLL = 4
QUARTERS = 4
S_Q = S // QUARTERS


@functools.partial(
    pl.kernel,
    mesh=_mesh,
    out_type=jax.ShapeDtypeStruct((B, HD2, S), jnp.float32),
    scratch_types=[
        pltpu.VMEM((S,), jnp.int32),
        pltpu.VMEM((2 * HD2,), jnp.float32),
        pltpu.VMEM((D_PER_W, S), jnp.float32),
        pltpu.SemaphoreType.DMA,
        pltpu.SemaphoreType.DMA,
        pltpu.SemaphoreType.DMA,
    ],
)
def _rope_theta_sc(pos_hbm, invf_hbm, out_hbm, pos_v, invf_v, out_v,
                   in_sem, invf_sem, out_sem):
    wid = lax.axis_index("s") * NUM_CORES + lax.axis_index("c")
    b = wid // NUM_SUBCORES
    wd = wid % NUM_SUBCORES
    d0 = wd * D_PER_W
    h_invf = pltpu.async_copy(invf_hbm, invf_v.at[pl.ds(0, HD2)], invf_sem)
    pos_handles = [
        pltpu.async_copy(pos_hbm.at[b, pl.ds(q * S_Q, S_Q)],
                         pos_v.at[pl.ds(q * S_Q, S_Q)], in_sem)
        for q in range(QUARTERS)
    ]
    h_invf.wait()
    fvr = invf_v[pl.ds(d0, L)]
    fs = [fvr[dd] for dd in range(D_PER_W)]

    out_handles = []
    for q in range(QUARTERS):
        pos_handles[q].wait()

        @plsc.parallel_loop(q * S_Q, (q + 1) * S_Q, step=L, unroll=UNROLL)
        def body(row):
            pf = pos_v[pl.ds(row, L)].astype(jnp.float32)
            for dd in range(D_PER_W):
                out_v[dd, pl.ds(row, L)] = pf * fs[dd]

        out_handles.append(pltpu.async_copy(
            out_v.at[:, pl.ds(q * S_Q, S_Q)],
            out_hbm.at[b, pl.ds(d0, D_PER_W), pl.ds(q * S_Q, S_Q)],
            out_sem))
    for h in out_handles:
        h.wait()


def kernel(position_ids, inv_freq):
    out_t = _rope_theta_sc(position_ids.astype(jnp.int32), inv_freq)
    return jnp.swapaxes(out_t, 1, 2)


# speedup vs baseline: 1.0158x; 1.0067x over previous
import functools

import jax
import jax.numpy as jnp
from jax import lax
from jax.experimental import pallas as pl
from jax.experimental.pallas import tpu as pltpu
from jax.experimental.pallas import tpu_sc as plsc

L = 16
NUM_CORES = 2
NUM_SUBCORES = 16
NW = NUM_CORES * NUM_SUBCORES
B = 2
S = 8192
HD2 = 64
D_PER_W = HD2 * B // NW

_mesh = plsc.VectorSubcoreMesh(core_axis_name="c", subcore_axis_name="s")


UNROLL = 4
QUARTERS = 4
S_Q = S // QUARTERS


@functools.partial(
    pl.kernel,
    mesh=_mesh,
    out_type=jax.ShapeDtypeStruct((B, HD2, S), jnp.float32),
    scratch_types=[
        pltpu.VMEM((S,), jnp.int32),
        pltpu.VMEM((2 * HD2,), jnp.float32),
        pltpu.VMEM((D_PER_W, S), jnp.float32),
        pltpu.SemaphoreType.DMA,
        pltpu.SemaphoreType.DMA,
        pltpu.SemaphoreType.DMA,
    ],
)
def _rope_theta_sc(pos_hbm, invf_hbm, out_hbm, pos_v, invf_v, out_v,
                   in_sem, invf_sem, out_sem):
    wid = lax.axis_index("s") * NUM_CORES + lax.axis_index("c")
    b = wid // NUM_SUBCORES
    wd = wid % NUM_SUBCORES
    d0 = wd * D_PER_W
    h_invf = pltpu.async_copy(invf_hbm, invf_v.at[pl.ds(0, HD2)], invf_sem)
    h_pos = pltpu.async_copy(pos_hbm.at[b], pos_v, in_sem)
    h_invf.wait()
    fvr = invf_v[pl.ds(d0, L)]
    fs = [fvr[dd] for dd in range(D_PER_W)]
    h_pos.wait()

    out_handles = []
    for q in range(QUARTERS):
        @plsc.parallel_loop(q * S_Q, (q + 1) * S_Q, step=L, unroll=UNROLL)
        def body(row):
            pf = pos_v[pl.ds(row, L)].astype(jnp.float32)
            for dd in range(D_PER_W):
                out_v[dd, pl.ds(row, L)] = pf * fs[dd]

        out_handles.append(pltpu.async_copy(
            out_v.at[:, pl.ds(q * S_Q, S_Q)],
            out_hbm.at[b, pl.ds(d0, D_PER_W), pl.ds(q * S_Q, S_Q)],
            out_sem))
    for h in out_handles:
        h.wait()


def kernel(position_ids, inv_freq):
    out_t = _rope_theta_sc(position_ids.astype(jnp.int32), inv_freq)
    return jnp.swapaxes(out_t, 1, 2)
